# Initial kernel scaffold; baseline (speedup 1.0000x reference)
import functools

import jax
import jax.numpy as jnp
from jax import lax
from jax.experimental import pallas as pl
from jax.experimental.pallas import tpu as pltpu
from jax.experimental.pallas import tpu_sc as plsc

_N = 10000
_E = 320000
_NFEAT = 128
_NHID = 64
_NCLASS = 40

_NC = 2
_NS = 16
_NT = _NC * _NS
_RPT = _N // _NS
_EPT = _E // _NT
_W = 80
_NWIN = _EPT // _W

_mesh = plsc.VectorSubcoreMesh(core_axis_name="c", subcore_axis_name="s")


@functools.partial(
    pl.kernel,
    out_type=jax.ShapeDtypeStruct((_NC, _N, 1), jnp.float32),
    mesh=_mesh,
    scratch_types=[
        pltpu.VMEM((1, _W), jnp.int32),
        pltpu.VMEM((_W,), jnp.float32),
        pltpu.VMEM_SHARED((_N,), jnp.float32),
    ],
)
def _deg_sc(row_hbm, zeros_hbm, out_hbm, idxb, ones_v, acc):
    c = lax.axis_index("c")
    s = lax.axis_index("s")
    one16 = jnp.ones((16,), jnp.float32)
    for i in range(_W // 16):
        ones_v[pl.ds(i * 16, 16)] = one16
    pltpu.sync_copy(zeros_hbm.at[pl.ds(s * _RPT, _RPT)],
                    acc.at[pl.ds(s * _RPT, _RPT)])
    plsc.subcore_barrier()
    base = (c * _NS + s) * _EPT

    @pl.loop(0, _NWIN)
    def _win(w):
        pltpu.sync_copy(row_hbm.at[pl.ds(base + w * _W, _W)], idxb.at[0])
        pltpu.sync_copy(ones_v, acc.at[idxb.at[0]], add=True)

    plsc.subcore_barrier()
    pltpu.sync_copy(acc.at[pl.ds(s * _RPT, _RPT)],
                    out_hbm.at[c, pl.ds(s * _RPT, _RPT), 0])


def _make_scatter(d):
    @functools.partial(
        pl.kernel,
        out_type=jax.ShapeDtypeStruct((_NC, _N, d), jnp.float32),
        mesh=_mesh,
        scratch_types=[
            pltpu.VMEM((1, _W), jnp.int32),
            pltpu.VMEM((1, _W), jnp.int32),
            pltpu.VMEM((1, _W, d), jnp.float32),
            pltpu.VMEM_SHARED((_N, d), jnp.float32),
            pltpu.SemaphoreType.DMA,
        ],
    )
    def _scatter_sc(tab_hbm, row_hbm, col_hbm, zeros_hbm, out_hbm,
                    ridx, cidx, rows, acc, gsem):
        c = lax.axis_index("c")
        s = lax.axis_index("s")
        pltpu.sync_copy(zeros_hbm.at[pl.ds(s * _RPT, _RPT)],
                        acc.at[pl.ds(s * _RPT, _RPT)])
        plsc.subcore_barrier()
        base = (c * _NS + s) * _EPT

        @pl.loop(0, _NWIN)
        def _win(w):
            eb = base + w * _W
            pltpu.sync_copy(row_hbm.at[pl.ds(eb, _W)], ridx.at[0])
            pltpu.sync_copy(col_hbm.at[pl.ds(eb, _W)], cidx.at[0])
            pltpu.async_copy(tab_hbm.at[ridx.at[0]], rows.at[0], gsem).wait()
            pltpu.sync_copy(rows.at[0], acc.at[cidx.at[0]], add=True)

        plsc.subcore_barrier()
        pltpu.sync_copy(acc.at[pl.ds(s * _RPT, _RPT)],
                        out_hbm.at[c, pl.ds(s * _RPT, _RPT)])

    return _scatter_sc


_scatter_hid = _make_scatter(_NHID)
_scatter_cls = _make_scatter(_NCLASS)


def _tc1_body(dp_ref, x_ref, w_ref, b_ref, h_ref, dis_ref):
    deg = 1.0 + dp_ref[0] + dp_ref[1]
    dis = lax.rsqrt(deg)
    h = jnp.dot(x_ref[...], w_ref[...],
                preferred_element_type=jnp.float32) + b_ref[...]
    h_ref[...] = dis * h
    dis_ref[...] = dis


def _tc2_body(a_ref, h_ref, d_ref, w_ref, b_ref, o_ref):
    dis = d_ref[...]
    u = dis * (a_ref[0] + a_ref[1] + h_ref[...])
    z = jnp.maximum(u, 0.0)
    h2 = jnp.dot(z, w_ref[...],
                 preferred_element_type=jnp.float32) + b_ref[...]
    o_ref[...] = dis * h2


def _tc3_body(a_ref, h_ref, d_ref, o_ref):
    u = d_ref[...] * (a_ref[0] + a_ref[1] + h_ref[...])
    m = jnp.max(u, axis=1, keepdims=True)
    lse = jnp.log(jnp.sum(jnp.exp(u - m), axis=1, keepdims=True)) + m
    o_ref[...] = u - lse


_tc1 = pl.pallas_call(
    _tc1_body,
    out_shape=(jax.ShapeDtypeStruct((_N, _NHID), jnp.float32),
               jax.ShapeDtypeStruct((_N, 1), jnp.float32)))
_tc2 = pl.pallas_call(
    _tc2_body, out_shape=jax.ShapeDtypeStruct((_N, _NCLASS), jnp.float32))
_tc3 = pl.pallas_call(
    _tc3_body, out_shape=jax.ShapeDtypeStruct((_N, _NCLASS), jnp.float32))


@jax.jit
def kernel(x, edge_index, W1, b1, W2, b2):
    row = edge_index[0].astype(jnp.int32)
    col = edge_index[1].astype(jnp.int32)
    zeros_n = jnp.zeros((_N,), jnp.float32)
    zeros_h = jnp.zeros((_N, _NHID), jnp.float32)
    zeros_c = jnp.zeros((_N, _NCLASS), jnp.float32)

    deg_parts = _deg_sc(row, zeros_n)
    h1p, dis = _tc1(deg_parts, x, W1.T, b1.reshape(1, -1))
    a1 = _scatter_hid(h1p, row, col, zeros_h)
    h2p = _tc2(a1, h1p, dis, W2.T, b2.reshape(1, -1))
    a2 = _scatter_cls(h2p, row, col, zeros_c)
    return _tc3(a2, h2p, dis)


# speedup vs baseline: 15.0572x; 15.0572x over previous
import functools

import jax
import jax.numpy as jnp
from jax import lax
from jax.experimental import pallas as pl
from jax.experimental.pallas import tpu as pltpu
from jax.experimental.pallas import tpu_sc as plsc

_N = 10000
_E = 320000
_NFEAT = 128
_NHID = 64
_NCLASS = 40

_NC = 2
_NS = 16
_NT = _NC * _NS
_RPT = _N // _NS
_EPT = _E // _NT
_W = 80
_NWIN = _EPT // _W

_mesh = plsc.VectorSubcoreMesh(core_axis_name="c", subcore_axis_name="s")


_NPAD = 10240
_RPT_PAD = _NPAD // _NS


@functools.partial(
    pl.kernel,
    out_type=jax.ShapeDtypeStruct((_NC, _NPAD), jnp.float32),
    mesh=_mesh,
    scratch_types=[
        pltpu.VMEM((1, _W), jnp.int32),
        pltpu.VMEM((_W,), jnp.float32),
        pltpu.VMEM_SHARED((_NPAD,), jnp.float32),
    ],
)
def _deg_sc(row_hbm, zeros_hbm, out_hbm, idxb, ones_v, acc):
    c = lax.axis_index("c")
    s = lax.axis_index("s")
    one16 = jnp.ones((16,), jnp.float32)
    for i in range(_W // 16):
        ones_v[pl.ds(i * 16, 16)] = one16
    pltpu.sync_copy(zeros_hbm.at[pl.ds(s * _RPT_PAD, _RPT_PAD)],
                    acc.at[pl.ds(s * _RPT_PAD, _RPT_PAD)])
    plsc.subcore_barrier()
    base = (c * _NS + s) * _EPT

    @pl.loop(0, _NWIN)
    def _win(w):
        pltpu.sync_copy(row_hbm.at[pl.ds(base + w * _W, _W)], idxb.at[0])
        pltpu.sync_copy(ones_v, acc.at[idxb.at[0]], add=True)

    plsc.subcore_barrier()
    pltpu.sync_copy(acc.at[pl.ds(s * _RPT_PAD, _RPT_PAD)],
                    out_hbm.at[c, pl.ds(s * _RPT_PAD, _RPT_PAD)])


def _make_scatter(d):
    @functools.partial(
        pl.kernel,
        out_type=jax.ShapeDtypeStruct((_NC, _NPAD, d), jnp.float32),
        mesh=_mesh,
        scratch_types=[
            pltpu.VMEM((1, _W), jnp.int32),
            pltpu.VMEM((1, _W), jnp.int32),
            pltpu.VMEM((1, _W, d), jnp.float32),
            pltpu.VMEM_SHARED((_NPAD, d), jnp.float32),
            pltpu.SemaphoreType.DMA,
        ],
        compiler_params=pltpu.CompilerParams(use_tc_tiling_on_sc=False),
    )
    def _scatter_sc(tab_hbm, row_hbm, col_hbm, zeros_hbm, out_hbm,
                    ridx, cidx, rows, acc, gsem):
        c = lax.axis_index("c")
        s = lax.axis_index("s")
        pltpu.sync_copy(zeros_hbm.at[pl.ds(s * _RPT_PAD, _RPT_PAD)],
                        acc.at[pl.ds(s * _RPT_PAD, _RPT_PAD)])
        plsc.subcore_barrier()
        base = (c * _NS + s) * _EPT

        @pl.loop(0, _NWIN)
        def _win(w):
            eb = base + w * _W
            pltpu.sync_copy(row_hbm.at[pl.ds(eb, _W)], ridx.at[0])
            pltpu.sync_copy(col_hbm.at[pl.ds(eb, _W)], cidx.at[0])
            pltpu.async_copy(tab_hbm.at[ridx.at[0]], rows.at[0], gsem).wait()
            pltpu.sync_copy(rows.at[0], acc.at[cidx.at[0]], add=True)

        plsc.subcore_barrier()
        pltpu.sync_copy(acc.at[pl.ds(s * _RPT_PAD, _RPT_PAD)],
                        out_hbm.at[c, pl.ds(s * _RPT_PAD, _RPT_PAD)])

    return _scatter_sc


_scatter_hid = _make_scatter(_NHID)
_scatter_cls = _make_scatter(_NCLASS)


def _tc1_body(dp_ref, x_ref, w_ref, b_ref, h_ref, dis_ref):
    deg = 1.0 + dp_ref[0] + dp_ref[1]
    dis = lax.rsqrt(deg)
    h = jnp.dot(x_ref[...], w_ref[...],
                preferred_element_type=jnp.float32) + b_ref[...]
    h_ref[...] = dis * h
    dis_ref[...] = dis


def _tc2_body(a_ref, h_ref, d_ref, w_ref, b_ref, o_ref):
    dis = d_ref[...]
    u = dis * (a_ref[0] + a_ref[1] + h_ref[...])
    z = jnp.maximum(u, 0.0)
    h2 = jnp.dot(z, w_ref[...],
                 preferred_element_type=jnp.float32) + b_ref[...]
    o_ref[...] = dis * h2


def _tc3_body(a_ref, h_ref, d_ref, o_ref):
    u = d_ref[...] * (a_ref[0] + a_ref[1] + h_ref[...])
    m = jnp.max(u, axis=1, keepdims=True)
    lse = jnp.log(jnp.sum(jnp.exp(u - m), axis=1, keepdims=True)) + m
    o_ref[...] = u - lse


_tc1 = pl.pallas_call(
    _tc1_body,
    out_shape=(jax.ShapeDtypeStruct((_N, _NHID), jnp.float32),
               jax.ShapeDtypeStruct((_N, 1), jnp.float32)))
_tc2 = pl.pallas_call(
    _tc2_body, out_shape=jax.ShapeDtypeStruct((_N, _NCLASS), jnp.float32))
_tc3 = pl.pallas_call(
    _tc3_body, out_shape=jax.ShapeDtypeStruct((_N, _NCLASS), jnp.float32))


@jax.jit
def kernel(x, edge_index, W1, b1, W2, b2):
    row = edge_index[0].astype(jnp.int32)
    col = edge_index[1].astype(jnp.int32)
    zeros_n = jnp.zeros((_NPAD,), jnp.float32)
    zeros_h = jnp.zeros((_NPAD, _NHID), jnp.float32)
    zeros_c = jnp.zeros((_NPAD, _NCLASS), jnp.float32)

    deg_parts = _deg_sc(row, zeros_n)[:, :_N, None]
    h1p, dis = _tc1(deg_parts, x, W1.T, b1.reshape(1, -1))
    a1 = _scatter_hid(h1p, row, col, zeros_h)[:, :_N]
    h2p = _tc2(a1, h1p, dis, W2.T, b2.reshape(1, -1))
    a2 = _scatter_cls(h2p, row, col, zeros_c)[:, :_N]
    return _tc3(a2, h2p, dis)


# speedup vs baseline: 43.3981x; 2.8822x over previous
import functools

import jax
import jax.numpy as jnp
from jax import lax
from jax.experimental import pallas as pl
from jax.experimental.pallas import tpu as pltpu
from jax.experimental.pallas import tpu_sc as plsc

_N = 10000
_E = 320000
_NFEAT = 128
_NHID = 64
_NCLASS = 40

_NC = 2
_NS = 16
_NT = _NC * _NS
_RPT = _N // _NS
_EPT = _E // _NT
_W = 80
_NWIN = _EPT // _W

_mesh = plsc.VectorSubcoreMesh(core_axis_name="c", subcore_axis_name="s")


_NPAD = 10240
_RPT_PAD = _NPAD // _NS
_DEPTH = 16


@functools.partial(
    pl.kernel,
    out_type=jax.ShapeDtypeStruct((_NC, _NPAD), jnp.float32),
    mesh=_mesh,
    scratch_types=[
        pltpu.VMEM((_NWIN, _W), jnp.int32),
        pltpu.VMEM((_W,), jnp.float32),
        pltpu.VMEM_SHARED((_NPAD,), jnp.float32),
        pltpu.SemaphoreType.DMA,
        pltpu.SemaphoreType.DMA,
    ],
)
def _deg_sc(row_hbm, zeros_hbm, out_hbm, idxb, ones_v, acc, ssem, isem):
    c = lax.axis_index("c")
    s = lax.axis_index("s")
    tid = c * _NS + s
    one16 = jnp.ones((16,), jnp.float32)
    for i in range(_W // 16):
        ones_v[pl.ds(i * 16, 16)] = one16
    iz = pltpu.async_copy(zeros_hbm.at[pl.ds(s * _RPT_PAD, _RPT_PAD)],
                          acc.at[pl.ds(s * _RPT_PAD, _RPT_PAD)], isem)
    ic = pltpu.async_copy(row_hbm.at[tid], idxb, ssem)
    ic.wait()
    iz.wait()
    plsc.subcore_barrier()

    @pl.loop(0, _NWIN)
    def _win(w):
        pltpu.async_copy(ones_v, acc.at[idxb.at[w]], ssem, add=True)

        @pl.when(w >= _DEPTH)
        def _drain():
            pltpu.make_async_copy(ones_v, acc.at[idxb.at[w]], ssem).wait()

    for _ in range(_DEPTH):
        pltpu.make_async_copy(ones_v, acc.at[idxb.at[0]], ssem).wait()
    plsc.subcore_barrier()
    pltpu.sync_copy(acc.at[pl.ds(s * _RPT_PAD, _RPT_PAD)],
                    out_hbm.at[c, pl.ds(s * _RPT_PAD, _RPT_PAD)])


_NB = 5
_NG = _NWIN // _NB


def _make_scatter(d):
    @functools.partial(
        pl.kernel,
        out_type=jax.ShapeDtypeStruct((_NC, _NPAD, d), jnp.float32),
        mesh=_mesh,
        scratch_types=[
            pltpu.VMEM((_NWIN, _W), jnp.int32),
            pltpu.VMEM((_NWIN, _W), jnp.int32),
            pltpu.VMEM((_NB, _W, d), jnp.float32),
            pltpu.VMEM_SHARED((_NPAD, d), jnp.float32),
        ] + [pltpu.SemaphoreType.DMA] * (2 * _NB + 1),
        compiler_params=pltpu.CompilerParams(use_tc_tiling_on_sc=False),
    )
    def _scatter_sc(tab_hbm, row_hbm, col_hbm, zeros_hbm, out_hbm,
                    ridx, cidx, rows, acc, *sems):
        gsem = sems[:_NB]
        ssem = sems[_NB:2 * _NB]
        isem = sems[2 * _NB]
        c = lax.axis_index("c")
        s = lax.axis_index("s")
        tid = c * _NS + s
        iz = pltpu.async_copy(zeros_hbm.at[pl.ds(s * _RPT_PAD, _RPT_PAD)],
                              acc.at[pl.ds(s * _RPT_PAD, _RPT_PAD)], isem)
        ir = pltpu.async_copy(row_hbm.at[tid], ridx, gsem[0])
        ic = pltpu.async_copy(col_hbm.at[tid], cidx, gsem[1])
        ir.wait()
        ic.wait()
        iz.wait()
        plsc.subcore_barrier()

        for b in range(_NB):
            pltpu.async_copy(tab_hbm.at[ridx.at[b]], rows.at[b], gsem[b])

        @pl.loop(0, _NG)
        def _grp(g):
            w0 = g * _NB
            for b in range(_NB):
                pltpu.make_async_copy(
                    tab_hbm.at[ridx.at[w0 + b]], rows.at[b], gsem[b]).wait()
                pltpu.async_copy(
                    rows.at[b], acc.at[cidx.at[w0 + b]], ssem[b], add=True)

            @pl.when(g + 1 < _NG)
            def _next():
                for b in range(_NB):
                    pltpu.make_async_copy(
                        rows.at[b], acc.at[cidx.at[w0 + b]], ssem[b]).wait()
                    pltpu.async_copy(
                        tab_hbm.at[ridx.at[w0 + _NB + b]], rows.at[b], gsem[b])

        w0 = (_NG - 1) * _NB
        for b in range(_NB):
            pltpu.make_async_copy(
                rows.at[b], acc.at[cidx.at[w0 + b]], ssem[b]).wait()
        plsc.subcore_barrier()
        pltpu.sync_copy(acc.at[pl.ds(s * _RPT_PAD, _RPT_PAD)],
                        out_hbm.at[c, pl.ds(s * _RPT_PAD, _RPT_PAD)])

    return _scatter_sc


_scatter_hid = _make_scatter(_NHID)
_scatter_cls = _make_scatter(_NCLASS)


def _tc1_body(dp_ref, x_ref, w_ref, b_ref, h_ref, dis_ref):
    deg = 1.0 + dp_ref[0] + dp_ref[1]
    dis = lax.rsqrt(deg)
    h = jnp.dot(x_ref[...], w_ref[...],
                preferred_element_type=jnp.float32) + b_ref[...]
    h_ref[...] = dis * h
    dis_ref[...] = dis


def _tc2_body(a_ref, h_ref, d_ref, w_ref, b_ref, o_ref):
    dis = d_ref[...]
    u = dis * (a_ref[0] + a_ref[1] + h_ref[...])
    z = jnp.maximum(u, 0.0)
    h2 = jnp.dot(z, w_ref[...],
                 preferred_element_type=jnp.float32) + b_ref[...]
    o_ref[...] = dis * h2


def _tc3_body(a_ref, h_ref, d_ref, o_ref):
    u = d_ref[...] * (a_ref[0] + a_ref[1] + h_ref[...])
    m = jnp.max(u, axis=1, keepdims=True)
    lse = jnp.log(jnp.sum(jnp.exp(u - m), axis=1, keepdims=True)) + m
    o_ref[...] = u - lse


_tc1 = pl.pallas_call(
    _tc1_body,
    out_shape=(jax.ShapeDtypeStruct((_N, _NHID), jnp.float32),
               jax.ShapeDtypeStruct((_N, 1), jnp.float32)))
_tc2 = pl.pallas_call(
    _tc2_body, out_shape=jax.ShapeDtypeStruct((_N, _NCLASS), jnp.float32))
_tc3 = pl.pallas_call(
    _tc3_body, out_shape=jax.ShapeDtypeStruct((_N, _NCLASS), jnp.float32))


@jax.jit
def kernel(x, edge_index, W1, b1, W2, b2):
    row = edge_index[0].astype(jnp.int32).reshape(_NT, _NWIN, _W)
    col = edge_index[1].astype(jnp.int32).reshape(_NT, _NWIN, _W)
    zeros_n = jnp.zeros((_NPAD,), jnp.float32)
    zeros_h = jnp.zeros((_NPAD, _NHID), jnp.float32)
    zeros_c = jnp.zeros((_NPAD, _NCLASS), jnp.float32)

    deg_parts = _deg_sc(row, zeros_n)[:, :_N, None]
    h1p, dis = _tc1(deg_parts, x, W1.T, b1.reshape(1, -1))
    a1 = _scatter_hid(h1p, row, col, zeros_h)[:, :_N]
    h2p = _tc2(a1, h1p, dis, W2.T, b2.reshape(1, -1))
    a2 = _scatter_cls(h2p, row, col, zeros_c)[:, :_N]
    return _tc3(a2, h2p, dis)


# speedup vs baseline: 45.3456x; 1.0449x over previous
import functools

import jax
import jax.numpy as jnp
from jax import lax
from jax.experimental import pallas as pl
from jax.experimental.pallas import tpu as pltpu
from jax.experimental.pallas import tpu_sc as plsc

_N = 10000
_E = 320000
_NFEAT = 128
_NHID = 64
_NCLASS = 40

_NC = 2
_NS = 16
_NT = _NC * _NS
_RPT = _N // _NS
_EPT = _E // _NT
_W = 40
_NWIN = _EPT // _W

_mesh = plsc.VectorSubcoreMesh(core_axis_name="c", subcore_axis_name="s")


_NPAD = 10240
_RPT_PAD = _NPAD // _NS
_LAST = _N - (_NS - 1) * _RPT_PAD
_DEPTH = 16


def _store_out(s, src, dst):
    @pl.when(s < _NS - 1)
    def _full():
        pltpu.sync_copy(src.at[pl.ds(s * _RPT_PAD, _RPT_PAD)],
                        dst.at[pl.ds(s * _RPT_PAD, _RPT_PAD)])

    @pl.when(s == _NS - 1)
    def _clip():
        pltpu.sync_copy(src.at[pl.ds((_NS - 1) * _RPT_PAD, _LAST)],
                        dst.at[pl.ds((_NS - 1) * _RPT_PAD, _LAST)])


@functools.partial(
    pl.kernel,
    out_type=jax.ShapeDtypeStruct((_NC, _NPAD), jnp.float32),
    mesh=_mesh,
    scratch_types=[
        pltpu.VMEM((_NWIN, _W), jnp.int32),
        pltpu.VMEM((48,), jnp.float32),
        pltpu.VMEM_SHARED((_NPAD,), jnp.float32),
        pltpu.SemaphoreType.DMA,
        pltpu.SemaphoreType.DMA,
    ],
)
def _deg_sc(ei_hbm, zeros_hbm, out_hbm, idxb, ones_v, acc, ssem, isem):
    c = lax.axis_index("c")
    s = lax.axis_index("s")
    tid = c * _NS + s
    one16 = jnp.ones((16,), jnp.float32)
    for i in range(3):
        ones_v[pl.ds(i * 16, 16)] = one16
    ones_w = ones_v.at[pl.ds(0, _W)]
    iz = pltpu.async_copy(zeros_hbm.at[pl.ds(s * _RPT_PAD, _RPT_PAD)],
                          acc.at[pl.ds(s * _RPT_PAD, _RPT_PAD)], isem)
    ic = pltpu.async_copy(ei_hbm.at[0, tid], idxb, ssem)
    ic.wait()
    iz.wait()
    plsc.subcore_barrier()

    @pl.loop(0, _NWIN)
    def _win(w):
        pltpu.async_copy(ones_w, acc.at[idxb.at[w]], ssem, add=True)

        @pl.when(w >= _DEPTH)
        def _drain():
            pltpu.make_async_copy(ones_w, acc.at[idxb.at[w]], ssem).wait()

    for _ in range(_DEPTH):
        pltpu.make_async_copy(ones_w, acc.at[idxb.at[0]], ssem).wait()
    plsc.subcore_barrier()
    pltpu.sync_copy(acc.at[pl.ds(s * _RPT_PAD, _RPT_PAD)],
                    out_hbm.at[c, pl.ds(s * _RPT_PAD, _RPT_PAD)])


_NB = 10
_NG = _NWIN // _NB


def _make_scatter(d):
    @functools.partial(
        pl.kernel,
        out_type=jax.ShapeDtypeStruct((_NC, _N, d), jnp.float32),
        mesh=_mesh,
        scratch_types=[
            pltpu.VMEM((_NWIN, _W), jnp.int32),
            pltpu.VMEM((_NWIN, _W), jnp.int32),
            pltpu.VMEM((_NB, _W, d), jnp.float32),
            pltpu.VMEM_SHARED((_NPAD, d), jnp.float32),
        ] + [pltpu.SemaphoreType.DMA] * (2 * _NB + 1),
        compiler_params=pltpu.CompilerParams(use_tc_tiling_on_sc=False),
    )
    def _scatter_sc(tab_hbm, ei_hbm, zeros_hbm, out_hbm,
                    ridx, cidx, rows, acc, *sems):
        gsem = sems[:_NB]
        ssem = sems[_NB:2 * _NB]
        isem = sems[2 * _NB]
        c = lax.axis_index("c")
        s = lax.axis_index("s")
        tid = c * _NS + s
        iz = pltpu.async_copy(zeros_hbm.at[pl.ds(s * _RPT_PAD, _RPT_PAD)],
                              acc.at[pl.ds(s * _RPT_PAD, _RPT_PAD)], isem)
        ir = pltpu.async_copy(ei_hbm.at[0, tid], ridx, gsem[0])
        ic = pltpu.async_copy(ei_hbm.at[1, tid], cidx, gsem[1])
        ir.wait()
        ic.wait()
        iz.wait()
        plsc.subcore_barrier()

        for b in range(_NB):
            pltpu.async_copy(tab_hbm.at[ridx.at[b]], rows.at[b], gsem[b])

        @pl.loop(0, _NG)
        def _grp(g):
            w0 = g * _NB
            for b in range(_NB):
                pltpu.make_async_copy(
                    tab_hbm.at[ridx.at[w0 + b]], rows.at[b], gsem[b]).wait()
                pltpu.async_copy(
                    rows.at[b], acc.at[cidx.at[w0 + b]], ssem[b], add=True)

            @pl.when(g + 1 < _NG)
            def _next():
                for b in range(_NB):
                    pltpu.make_async_copy(
                        rows.at[b], acc.at[cidx.at[w0 + b]], ssem[b]).wait()
                    pltpu.async_copy(
                        tab_hbm.at[ridx.at[w0 + _NB + b]], rows.at[b], gsem[b])

        w0 = (_NG - 1) * _NB
        for b in range(_NB):
            pltpu.make_async_copy(
                rows.at[b], acc.at[cidx.at[w0 + b]], ssem[b]).wait()
        plsc.subcore_barrier()
        _store_out(s, acc, out_hbm.at[c])

    return _scatter_sc


_scatter_hid = _make_scatter(_NHID)
_scatter_cls = _make_scatter(_NCLASS)


def _tc1_body(dp_ref, x_ref, w_ref, b_ref, h_ref, dis_ref):
    deg = 1.0 + dp_ref[0] + dp_ref[1]
    dis = lax.rsqrt(deg)
    h = jnp.dot(x_ref[...], w_ref[...],
                preferred_element_type=jnp.float32) + b_ref[...]
    h_ref[...] = dis * h
    dis_ref[...] = dis


def _tc2_body(a_ref, h_ref, d_ref, w_ref, b_ref, o_ref):
    dis = d_ref[...]
    u = dis * (a_ref[0] + a_ref[1] + h_ref[...])
    z = jnp.maximum(u, 0.0)
    h2 = jnp.dot(z, w_ref[...],
                 preferred_element_type=jnp.float32) + b_ref[...]
    o_ref[...] = dis * h2


def _tc3_body(a_ref, h_ref, d_ref, o_ref):
    u = d_ref[...] * (a_ref[0] + a_ref[1] + h_ref[...])
    m = jnp.max(u, axis=1, keepdims=True)
    lse = jnp.log(jnp.sum(jnp.exp(u - m), axis=1, keepdims=True)) + m
    o_ref[...] = u - lse


_tc1 = pl.pallas_call(
    _tc1_body,
    out_shape=(jax.ShapeDtypeStruct((_N, _NHID), jnp.float32),
               jax.ShapeDtypeStruct((_N, 1), jnp.float32)))
_tc2 = pl.pallas_call(
    _tc2_body, out_shape=jax.ShapeDtypeStruct((_N, _NCLASS), jnp.float32))
_tc3 = pl.pallas_call(
    _tc3_body, out_shape=jax.ShapeDtypeStruct((_N, _NCLASS), jnp.float32))


@jax.jit
def kernel(x, edge_index, W1, b1, W2, b2):
    ei4 = edge_index.astype(jnp.int32).reshape(2, _NT, _NWIN, _W)
    zeros_n = jnp.zeros((_NPAD,), jnp.float32)
    zeros_h = jnp.zeros((_NPAD, _NHID), jnp.float32)
    zeros_c = jnp.zeros((_NPAD, _NCLASS), jnp.float32)

    deg_parts = _deg_sc(ei4, zeros_n)[:, :_N, None]
    h1p, dis = _tc1(deg_parts, x, W1.T, b1.reshape(1, -1))
    a1 = _scatter_hid(h1p, ei4, zeros_h)
    h2p = _tc2(a1, h1p, dis, W2.T, b2.reshape(1, -1))
    a2 = _scatter_cls(h2p, ei4, zeros_c)
    return _tc3(a2, h2p, dis)


# speedup vs baseline: 47.3026x; 1.0432x over previous
import functools

import jax
import jax.numpy as jnp
from jax import lax
from jax.experimental import pallas as pl
from jax.experimental.pallas import tpu as pltpu
from jax.experimental.pallas import tpu_sc as plsc

_N = 10000
_E = 320000
_NFEAT = 128
_NHID = 64
_NCLASS = 40

_NC = 2
_NS = 16
_NT = _NC * _NS
_RPT = _N // _NS
_EPT = _E // _NT
_W = 80
_NWIN = _EPT // _W

_mesh = plsc.VectorSubcoreMesh(core_axis_name="c", subcore_axis_name="s")


_NPAD = 10240
_RPT_PAD = _NPAD // _NS
_LAST = _N - (_NS - 1) * _RPT_PAD
_DEPTH = 16


def _store_out(s, src, dst):
    @pl.when(s < _NS - 1)
    def _full():
        pltpu.sync_copy(src.at[pl.ds(s * _RPT_PAD, _RPT_PAD)],
                        dst.at[pl.ds(s * _RPT_PAD, _RPT_PAD)])

    @pl.when(s == _NS - 1)
    def _clip():
        pltpu.sync_copy(src.at[pl.ds((_NS - 1) * _RPT_PAD, _LAST)],
                        dst.at[pl.ds((_NS - 1) * _RPT_PAD, _LAST)])


@functools.partial(
    pl.kernel,
    out_type=jax.ShapeDtypeStruct((_NC, _NPAD), jnp.float32),
    mesh=_mesh,
    scratch_types=[
        pltpu.VMEM((_NWIN, _W), jnp.int32),
        pltpu.VMEM((_W,), jnp.float32),
        pltpu.VMEM_SHARED((_NPAD,), jnp.float32),
        pltpu.SemaphoreType.DMA,
        pltpu.SemaphoreType.DMA,
    ],
)
def _deg_sc(ei_hbm, zeros_hbm, out_hbm, idxb, ones_v, acc, ssem, isem):
    c = lax.axis_index("c")
    s = lax.axis_index("s")
    tid = c * _NS + s
    one16 = jnp.ones((16,), jnp.float32)
    for i in range(_W // 16):
        ones_v[pl.ds(i * 16, 16)] = one16
    ones_w = ones_v
    iz = pltpu.async_copy(zeros_hbm.at[pl.ds(s * _RPT_PAD, _RPT_PAD)],
                          acc.at[pl.ds(s * _RPT_PAD, _RPT_PAD)], isem)
    ic = pltpu.async_copy(ei_hbm.at[0, tid], idxb, ssem)
    ic.wait()
    iz.wait()
    plsc.subcore_barrier()

    @pl.loop(0, _NWIN)
    def _win(w):
        pltpu.async_copy(ones_w, acc.at[idxb.at[w]], ssem, add=True)

        @pl.when(w >= _DEPTH)
        def _drain():
            pltpu.make_async_copy(ones_w, acc.at[idxb.at[w]], ssem).wait()

    for _ in range(_DEPTH):
        pltpu.make_async_copy(ones_w, acc.at[idxb.at[0]], ssem).wait()
    plsc.subcore_barrier()
    pltpu.sync_copy(acc.at[pl.ds(s * _RPT_PAD, _RPT_PAD)],
                    out_hbm.at[c, pl.ds(s * _RPT_PAD, _RPT_PAD)])


_NB = 5
_NG = _NWIN // _NB


def _make_scatter(d):
    @functools.partial(
        pl.kernel,
        out_type=jax.ShapeDtypeStruct((_NC, _N, d), jnp.float32),
        mesh=_mesh,
        scratch_types=[
            pltpu.VMEM((_NWIN, _W), jnp.int32),
            pltpu.VMEM((_NWIN, _W), jnp.int32),
            pltpu.VMEM((_NB, _W, d), jnp.float32),
            pltpu.VMEM_SHARED((_NPAD, d), jnp.float32),
        ] + [pltpu.SemaphoreType.DMA] * (2 * _NB + 1),
        compiler_params=pltpu.CompilerParams(use_tc_tiling_on_sc=False),
    )
    def _scatter_sc(tab_hbm, ei_hbm, zeros_hbm, out_hbm,
                    ridx, cidx, rows, acc, *sems):
        gsem = sems[:_NB]
        ssem = sems[_NB:2 * _NB]
        isem = sems[2 * _NB]
        c = lax.axis_index("c")
        s = lax.axis_index("s")
        tid = c * _NS + s
        iz = pltpu.async_copy(zeros_hbm.at[pl.ds(s * _RPT_PAD, _RPT_PAD)],
                              acc.at[pl.ds(s * _RPT_PAD, _RPT_PAD)], isem)
        ir = pltpu.async_copy(ei_hbm.at[0, tid], ridx, gsem[0])
        ic = pltpu.async_copy(ei_hbm.at[1, tid], cidx, gsem[1])
        ir.wait()
        ic.wait()
        iz.wait()
        plsc.subcore_barrier()

        for b in range(_NB):
            pltpu.async_copy(tab_hbm.at[ridx.at[b]], rows.at[b], gsem[b])

        @pl.loop(0, _NG)
        def _grp(g):
            w0 = g * _NB
            for b in range(_NB):
                pltpu.make_async_copy(
                    tab_hbm.at[ridx.at[w0 + b]], rows.at[b], gsem[b]).wait()
                pltpu.async_copy(
                    rows.at[b], acc.at[cidx.at[w0 + b]], ssem[b], add=True)

            @pl.when(g + 1 < _NG)
            def _next():
                for b in range(_NB):
                    pltpu.make_async_copy(
                        rows.at[b], acc.at[cidx.at[w0 + b]], ssem[b]).wait()
                    pltpu.async_copy(
                        tab_hbm.at[ridx.at[w0 + _NB + b]], rows.at[b], gsem[b])

        w0 = (_NG - 1) * _NB
        for b in range(_NB):
            pltpu.make_async_copy(
                rows.at[b], acc.at[cidx.at[w0 + b]], ssem[b]).wait()
        plsc.subcore_barrier()
        _store_out(s, acc, out_hbm.at[c])

    return _scatter_sc


_scatter_hid = _make_scatter(_NHID)
_scatter_cls = _make_scatter(_NCLASS)


def _mm_body(x_ref, w_ref, o_ref):
    o_ref[...] = jnp.dot(x_ref[...], w_ref[...],
                         preferred_element_type=jnp.float32)


def _lsm_body(u_ref, o_ref):
    u = u_ref[...]
    m = jnp.max(u, axis=1, keepdims=True)
    lse = jnp.log(jnp.sum(jnp.exp(u - m), axis=1, keepdims=True)) + m
    o_ref[...] = u - lse


_mm1 = pl.pallas_call(
    _mm_body, out_shape=jax.ShapeDtypeStruct((_N, _NHID), jnp.float32))
_mm2 = pl.pallas_call(
    _mm_body, out_shape=jax.ShapeDtypeStruct((_N, _NCLASS), jnp.float32))
_lsm = pl.pallas_call(
    _lsm_body, out_shape=jax.ShapeDtypeStruct((_N, _NCLASS), jnp.float32))


@jax.jit
def kernel(x, edge_index, W1, b1, W2, b2):
    ei4 = edge_index.astype(jnp.int32).reshape(2, _NT, _NWIN, _W)
    zeros_n = jnp.zeros((_NPAD,), jnp.float32)
    zeros_h = jnp.zeros((_NPAD, _NHID), jnp.float32)
    zeros_c = jnp.zeros((_NPAD, _NCLASS), jnp.float32)

    deg_parts = _deg_sc(ei4, zeros_n)
    dis = lax.rsqrt(1.0 + deg_parts[0, :_N] + deg_parts[1, :_N])[:, None]
    h1 = _mm1(x, W1.T)
    h1p = dis * (h1 + b1)
    a1 = _scatter_hid(h1p, ei4, zeros_h)
    z = jnp.maximum(dis * (a1[0] + a1[1] + h1p), 0.0)
    h2 = _mm2(z, W2.T)
    h2p = dis * (h2 + b2)
    a2 = _scatter_cls(h2p, ei4, zeros_c)
    u = dis * (a2[0] + a2[1] + h2p)
    return _lsm(u)
